# baseline (speedup 1.0000x reference)
import functools

import jax
import jax.numpy as jnp
from jax import lax
from jax.experimental import pallas as pl
from jax.experimental.pallas import tpu as pltpu
from jax.experimental.pallas import tpu_sc as plsc

_SC_INFO = plsc.get_sparse_core_info()
_NC = _SC_INFO.num_cores
_NS = _SC_INFO.num_subcores
_NW = _NC * _NS

_CHUNK = 16
_NBUF = 3


def _make_sc_copy(S, D, dtype):
    rows_per_w = S // _NW
    nsteps = rows_per_w // _CHUNK

    mesh = plsc.VectorSubcoreMesh(core_axis_name="c", subcore_axis_name="s")

    @functools.partial(
        pl.kernel,
        mesh=mesh,
        out_type=jax.ShapeDtypeStruct((1, S, D), dtype),
        scratch_types=[
            pltpu.VMEM((_NBUF, _CHUNK, D), dtype),
            pltpu.SemaphoreType.DMA((_NBUF,)),
            pltpu.SemaphoreType.DMA((_NBUF,)),
        ],
    )
    def sc_copy(w_hbm, o_hbm, buf, in_sem, out_sem):
        wid = lax.axis_index("s") * _NC + lax.axis_index("c")
        base = wid * rows_per_w

        def in_copy(step, slot):
            return pltpu.make_async_copy(
                w_hbm.at[pl.ds(base + step * _CHUNK, _CHUNK)],
                buf.at[slot],
                in_sem.at[slot],
            )

        def out_copy(step, slot):
            return pltpu.make_async_copy(
                buf.at[slot],
                o_hbm.at[0, pl.ds(base + step * _CHUNK, _CHUNK)],
                out_sem.at[slot],
            )

        for s in range(min(_NBUF, nsteps)):
            in_copy(s, s).start()
        for step in range(nsteps):
            slot = step % _NBUF
            in_copy(step, slot).wait()
            out_copy(step, slot).start()
            nxt = step + _NBUF
            if nxt < nsteps:
                out_copy(step, slot).wait()
                in_copy(nxt, slot).start()
        for step in range(max(nsteps - _NBUF, 0), nsteps):
            out_copy(step, step % _NBUF).wait()

    return sc_copy


def kernel(embedding_weight, seq_len):
    del seq_len
    S, D = embedding_weight.shape
    return _make_sc_copy(S, D, embedding_weight.dtype)(embedding_weight)


# speedup vs baseline: 1.0090x; 1.0090x over previous
import functools

import jax
import jax.numpy as jnp
from jax import lax
from jax.experimental import pallas as pl
from jax.experimental.pallas import tpu as pltpu
from jax.experimental.pallas import tpu_sc as plsc

_SC_INFO = plsc.get_sparse_core_info()
_NC = _SC_INFO.num_cores
_NS = _SC_INFO.num_subcores
_NW = _NC * _NS

_CHUNK = 16
_NBUF = 3


def _make_sc_copy(S, D, dtype):
    rows_per_w = S // _NW
    nsteps = rows_per_w // _CHUNK

    mesh = plsc.VectorSubcoreMesh(core_axis_name="c", subcore_axis_name="s")

    @functools.partial(
        pl.kernel,
        mesh=mesh,
        out_type=jax.ShapeDtypeStruct((1, S, D), dtype),
        scratch_types=[
            pltpu.VMEM((_NBUF, _CHUNK, D), dtype),
            pltpu.SemaphoreType.DMA((_NBUF,)),
            pltpu.SemaphoreType.DMA((_NBUF,)),
        ],
    )
    def sc_copy(w_hbm, o_hbm, buf, in_sem, out_sem):
        wid = lax.axis_index("s") * _NC + lax.axis_index("c")
        base = wid * _CHUNK
        stride = _NW * _CHUNK

        def in_copy(step, slot):
            return pltpu.make_async_copy(
                w_hbm.at[pl.ds(base + step * stride, _CHUNK)],
                buf.at[slot],
                in_sem.at[slot],
            )

        def out_copy(step, slot):
            return pltpu.make_async_copy(
                buf.at[slot],
                o_hbm.at[0, pl.ds(base + step * stride, _CHUNK)],
                out_sem.at[slot],
            )

        for s in range(min(_NBUF, nsteps)):
            in_copy(s, s).start()
        for step in range(nsteps):
            slot = step % _NBUF
            in_copy(step, slot).wait()
            out_copy(step, slot).start()
            nxt = step + _NBUF
            if nxt < nsteps:
                out_copy(step, slot).wait()
                in_copy(nxt, slot).start()
        for step in range(max(nsteps - _NBUF, 0), nsteps):
            out_copy(step, step % _NBUF).wait()

    return sc_copy


def kernel(embedding_weight, seq_len):
    del seq_len
    S, D = embedding_weight.shape
    return _make_sc_copy(S, D, embedding_weight.dtype)(embedding_weight)
